# baseline (speedup 1.0000x reference)
import functools

import jax
import jax.numpy as jnp
from jax import lax
from jax.experimental import pallas as pl
from jax.experimental.pallas import tpu as pltpu
from jax.experimental.pallas import tpu_sc as plsc

D = 768
T = 77
TPAD = 80
B = 256

NC = 2
NS = 16
NW = NC * NS
SEQ_PER_W = B // NW
LANES = 16
GROUPS = D // LANES

_TCHUNKS = [(0, 40, 40), (40, 32, 32), (72, 8, 5)]
_BUFROWS = max(sz for _, sz, _ in _TCHUNKS)


def _make_kernel():
    mesh = plsc.VectorSubcoreMesh(core_axis_name="c", subcore_axis_name="s")

    @functools.partial(
        pl.kernel,
        mesh=mesh,
        out_type=(
            jax.ShapeDtypeStruct((B, T, D), jnp.float32),
            jax.ShapeDtypeStruct((B, 8, D), jnp.float32),
        ),
        scratch_types=[
            pltpu.VMEM((SEQ_PER_W * TPAD,), jnp.int32),
            pltpu.VMEM((T, D), jnp.float32),
            pltpu.VMEM((_BUFROWS, D), jnp.float32),
            pltpu.VMEM((_BUFROWS, D), jnp.float32),
            pltpu.SemaphoreType.DMA,
            pltpu.SemaphoreType.DMA,
            pltpu.SemaphoreType.DMA,
            pltpu.SemaphoreType.DMA,
        ],
    )
    def k(tokens_hbm, table_hbm, pos_hbm, out_hbm, tail_hbm,
          idx_v, pos_v, rows_a, rows_b, sg0, sg1, sw0, sw1):
        wid = lax.axis_index("s") * NC + lax.axis_index("c")
        seq0 = wid * SEQ_PER_W
        pltpu.sync_copy(tokens_hbm.at[pl.ds(seq0 * TPAD, SEQ_PER_W * TPAD)],
                        idx_v)
        pltpu.sync_copy(pos_hbm, pos_v)

        bufs = (rows_a, rows_b)
        sems_g = (sg0, sg1)
        sems_w = (sw0, sw1)
        chunks = [(s,) + tc for s in range(SEQ_PER_W) for tc in _TCHUNKS]
        n = len(chunks)
        gathers = {}
        writes = {}

        def issue_gather(ci):
            s, t0, sz, _ = chunks[ci]
            gathers[ci] = pltpu.async_copy(
                table_hbm.at[idx_v.at[pl.ds(s * TPAD + t0, sz)]],
                bufs[ci % 2].at[pl.ds(0, sz)],
                sems_g[ci % 2],
            )

        issue_gather(0)
        for ci, (s, t0, sz, nadd) in enumerate(chunks):
            buf = bufs[ci % 2]
            gathers[ci].wait()
            if ci >= 1:
                writes[ci - 1].wait()
            if ci + 1 < n:
                issue_gather(ci + 1)

            def body(r, _):
                for g in range(GROUPS):
                    sl = pl.ds(g * LANES, LANES)
                    plsc.addupdate(buf.at[r, sl], pos_v[t0 + r, sl])
                return 0

            lax.fori_loop(0, nadd, body, 0)
            if t0 < 72:
                dst = out_hbm.at[seq0 + s, pl.ds(t0, sz)]
            else:
                dst = tail_hbm.at[seq0 + s]
            writes[ci] = pltpu.async_copy(
                buf.at[pl.ds(0, sz)], dst, sems_w[ci % 2],
            )
        writes[n - 1].wait()

    return k


_grid_kernel = _make_kernel()


def kernel(tokens, token_embedding, position_embedding):
    tok = jnp.pad(tokens.astype(jnp.int32), ((0, 0), (0, TPAD - T)))
    out, tail = _grid_kernel(tok.reshape(-1), token_embedding,
                             position_embedding)
    return lax.dynamic_update_slice(out, tail[:, :T - 72, :], (0, 72, 0))


# speedup vs baseline: 2.3659x; 2.3659x over previous
import functools

import jax
import jax.numpy as jnp
from jax import lax
from jax.experimental import pallas as pl
from jax.experimental.pallas import tpu as pltpu
from jax.experimental.pallas import tpu_sc as plsc

D = 768
T = 77
B = 256

NC = 2
NS = 16
NW = NC * NS
BBLK = B // NW
ROWS_PER_W = T * BBLK
TC0 = 5
LANES = 16
GROUPS = D // LANES

_TCHUNKS = [(i * TC0, TC0) for i in range(T // TC0)]
if T % TC0:
    _TCHUNKS.append((T - T % TC0, T % TC0))


def _make_kernel():
    mesh = plsc.VectorSubcoreMesh(core_axis_name="c", subcore_axis_name="s")

    @functools.partial(
        pl.kernel,
        mesh=mesh,
        out_type=jax.ShapeDtypeStruct((T, B, D), jnp.float32),
        scratch_types=[
            pltpu.VMEM((ROWS_PER_W,), jnp.int32),
            pltpu.VMEM((T, D), jnp.float32),
            pltpu.VMEM((TC0 * BBLK, D), jnp.float32),
            pltpu.VMEM((TC0 * BBLK, D), jnp.float32),
            pltpu.SemaphoreType.DMA,
            pltpu.SemaphoreType.DMA,
            pltpu.SemaphoreType.DMA,
            pltpu.SemaphoreType.DMA,
        ],
    )
    def k(tokens_hbm, table_hbm, pos_hbm, out_hbm,
          idx_v, pos_v, rows_a, rows_b, sg0, sg1, sw0, sw1):
        wid = lax.axis_index("s") * NC + lax.axis_index("c")
        b0 = wid * BBLK
        pltpu.sync_copy(tokens_hbm.at[pl.ds(wid * ROWS_PER_W, ROWS_PER_W)],
                        idx_v)
        pltpu.sync_copy(pos_hbm, pos_v)

        bufs = (rows_a, rows_b)
        sems_g = (sg0, sg1)
        sems_w = (sw0, sw1)
        n = len(_TCHUNKS)
        gathers = {}
        writes = {}

        def issue_gather(ci):
            t0, tc = _TCHUNKS[ci]
            gathers[ci] = pltpu.async_copy(
                table_hbm.at[idx_v.at[pl.ds(t0 * BBLK, tc * BBLK)]],
                bufs[ci % 2].at[pl.ds(0, tc * BBLK)],
                sems_g[ci % 2],
            )

        issue_gather(0)
        for ci, (t0, tc) in enumerate(_TCHUNKS):
            buf = bufs[ci % 2]
            gathers[ci].wait()
            if ci >= 1:
                for w in writes[ci - 1]:
                    w.wait()
            if ci + 1 < n:
                issue_gather(ci + 1)

            def body(j, _):
                r0 = j * BBLK

                def gbody(g, _):
                    sl = pl.ds(g * LANES, LANES)
                    pv = pos_v[t0 + j, sl]
                    for jj in range(BBLK):
                        plsc.addupdate(buf.at[r0 + jj, sl], pv)
                    return 0

                lax.fori_loop(0, GROUPS, gbody, 0)
                return 0

            lax.fori_loop(0, tc, body, 0)
            writes[ci] = [
                pltpu.async_copy(
                    buf.at[pl.ds(j * BBLK, BBLK)],
                    out_hbm.at[t0 + j, pl.ds(b0, BBLK)],
                    sems_w[ci % 2],
                )
                for j in range(tc)
            ]
        for w in writes[n - 1]:
            w.wait()

    return k


_grid_kernel = _make_kernel()


def kernel(tokens, token_embedding, position_embedding):
    tok = (tokens.astype(jnp.int32).T
           .reshape(T, NW, BBLK).transpose(1, 0, 2).reshape(-1))
    out = _grid_kernel(tok, token_embedding, position_embedding)
    return out.transpose(1, 0, 2)
